# baseline (speedup 1.0000x reference)
import functools

import jax
import jax.numpy as jnp
from jax import lax
from jax.experimental import pallas as pl
from jax.experimental.pallas import tpu as pltpu
from jax.experimental.pallas import tpu_sc as plsc

B = 16384
D = 64
H = 128
NU = 1000000
NCOMM = 100000

NC = 2
NS = 16
NW = NC * NS
BPW = B // NW
QTR = BPW // 4
HALF = BPW // 2

UBLK = NU // 128
CBLK = NCOMM // 128
UPAIR = (UBLK // NW) // 2
UREM = UBLK - UPAIR * 2 * NW
CPAIR = (CBLK // NW) // 2
CREM = CBLK - CPAIR * 2 * NW


@functools.lru_cache(maxsize=None)
def _build_sc_convert():
    mesh = plsc.VectorSubcoreMesh(core_axis_name="c", subcore_axis_name="s")

    @functools.partial(
        pl.kernel,
        mesh=mesh,
        out_type=(
            jax.ShapeDtypeStruct((NU // 2, 2 * D), jnp.float32),
            jax.ShapeDtypeStruct((NCOMM // 2, 2 * D), jnp.float32),
        ),
        scratch_types=[
            pltpu.VMEM((D, 2 * D), jnp.float32),
            pltpu.VMEM((D, 2 * D), jnp.float32),
            pltpu.VMEM((D, 2 * D), jnp.float32),
            pltpu.VMEM((D, 2 * D), jnp.float32),
            pltpu.SemaphoreType.DMA,
            pltpu.SemaphoreType.DMA,
            pltpu.SemaphoreType.DMA,
            pltpu.SemaphoreType.DMA,
        ],
        compiler_params=pltpu.CompilerParams(needs_layout_passes=False),
    )
    def sc_convert(utabT_hbm, ctabT_hbm, utail_hbm, ctail_hbm,
                   utab2_hbm, ctab2_hbm,
                   slab0, slab1, tout0, tout1, semr0, semr1, semw0, semw1):
        wid = lax.axis_index("s") * NC + lax.axis_index("c")
        lane = jax.lax.iota(jnp.int32, 16)

        def transpose(slab, tout):
            def body(f4, _):
                for ff in range(4):
                    for l in range(8):
                        fvec = f4 * 4 + ff
                        src = slab[fvec, pl.ds(16 * l, 16)]
                        row16 = 16 * l + lane
                        plsc.store_scatter(
                            tout,
                            [lax.shift_right_logical(row16, 1),
                             lax.bitwise_and(row16, 1) * D + fvec],
                            src)
                return _
            lax.fori_loop(0, D // 4, body, None)

        for tabT_hbm, tab2_hbm, npair, nrem, base0 in (
                (utabT_hbm, utab2_hbm, UPAIR, UREM, UPAIR * 2 * NW),
                (ctabT_hbm, ctab2_hbm, CPAIR, CREM, CPAIR * 2 * NW)):

            def rd(b, slab, semr, tabT_hbm=tabT_hbm):
                return pltpu.async_copy(
                    tabT_hbm.at[:, pl.ds(b * 128, 128)], slab, semr)

            def wr(b, tout, semw, tab2_hbm=tab2_hbm):
                return pltpu.async_copy(
                    tout, tab2_hbm.at[pl.ds(b * D, D)], semw)

            def drain(slab_or_tout, sem, tabT_hbm=tabT_hbm):
                pltpu.make_async_copy(
                    tabT_hbm.at[:, pl.ds(0, 128)], slab_or_tout, sem).wait()

            rd(wid * 2, slab0, semr0)

            def body(j, _, tabT_hbm=tabT_hbm, tab2_hbm=tab2_hbm, npair=npair):
                b0 = (wid + j * NW) * 2
                b1 = b0 + 1
                rd(b1, slab1, semr1)
                drain(slab0, semr0)

                @pl.when(j > 0)
                def _w0():
                    drain(tout0, semw0)

                transpose(slab0, tout0)
                wr(b0, tout0, semw0)

                @pl.when(j + 1 < npair)
                def _r0():
                    rd((wid + (j + 1) * NW) * 2, slab0, semr0)

                drain(slab1, semr1)

                @pl.when(j > 0)
                def _w1():
                    drain(tout1, semw1)

                transpose(slab1, tout1)
                wr(b1, tout1, semw1)
                return _

            lax.fori_loop(0, npair, body, None)
            drain(tout0, semw0)
            drain(tout1, semw1)

            @pl.when(wid < nrem)
            def _rem():
                b = base0 + wid
                rd(b, slab0, semr0)
                drain(slab0, semr0)
                transpose(slab0, tout0)
                wr(b, tout0, semw0)
                drain(tout0, semw0)

        @pl.when(wid == 0)
        def _ut():
            pltpu.sync_copy(utail_hbm, slab0.at[pl.ds(0, 32)])
            pltpu.sync_copy(slab0.at[pl.ds(0, 32)],
                            utab2_hbm.at[pl.ds(NU // 2 - 32, 32)])

        @pl.when(wid == 1)
        def _ct():
            pltpu.sync_copy(ctail_hbm, slab0.at[pl.ds(0, 16)])
            pltpu.sync_copy(slab0.at[pl.ds(0, 16)],
                            ctab2_hbm.at[pl.ds(NCOMM // 2 - 16, 16)])

    return sc_convert


@functools.lru_cache(maxsize=None)
def _build_sc_gather():
    mesh = plsc.VectorSubcoreMesh(core_axis_name="c", subcore_axis_name="s")

    @functools.partial(
        pl.kernel,
        mesh=mesh,
        out_type=(
            jax.ShapeDtypeStruct((B, D), jnp.float32),
            jax.ShapeDtypeStruct((B, D), jnp.float32),
        ),
        scratch_types=[
            pltpu.VMEM((BPW,), jnp.int32),
            pltpu.VMEM((QTR, 2 * D), jnp.float32),
            pltpu.VMEM((HALF, D), jnp.float32),
            pltpu.SemaphoreType.DMA,
        ],
    )
    def sc_gather(uidx_hbm, cidx_hbm, utab_hbm, ctab_hbm, uout_hbm, cout_hbm,
                  idx_v, pair_v, out_v, sem):
        wid = lax.axis_index("s") * NC + lax.axis_index("c")
        base = wid * BPW

        for idx_hbm, tab_hbm, out_hbm in ((uidx_hbm, utab_hbm, uout_hbm),
                                          (cidx_hbm, ctab_hbm, cout_hbm)):
            pltpu.sync_copy(idx_hbm.at[pl.ds(base, BPW)], idx_v)
            for q in range(4):

                def fire(g, _, tab_hbm=tab_hbm, q=q):
                    vec = idx_v[pl.ds(q * QTR + g * 16, 16)]
                    pvec = lax.shift_right_logical(vec, 1)
                    for k in range(16):
                        pltpu.async_copy(
                            tab_hbm.at[pvec[k]],
                            pair_v.at[g * 16 + k], sem)
                    return _

                lax.fori_loop(0, QTR // 16, fire, None)

                def drain(r, _, tab_hbm=tab_hbm):
                    pltpu.make_async_copy(
                        tab_hbm.at[0], pair_v.at[r], sem).wait()
                    return _

                lax.fori_loop(0, QTR, drain, None)

                def select(g, _, q=q):
                    vec = idx_v[pl.ds(q * QTR + g * 16, 16)]
                    hvec = lax.bitwise_and(vec, 1) * D
                    for k in range(16):
                        r = g * 16 + k
                        o = (q % 2) * QTR + g * 16 + k
                        h0 = hvec[k]
                        for c in range(D // 16):
                            out_v[o, pl.ds(c * 16, 16)] = (
                                pair_v[r, pl.ds(h0 + c * 16, 16)])
                    return _

                lax.fori_loop(0, QTR // 16, select, None)
                if q % 2 == 1:
                    pltpu.sync_copy(
                        out_v,
                        out_hbm.at[pl.ds(base + (q // 2) * HALF, HALF)])

    return sc_gather


BM = 2048


def _mlp_body(ue_r, ce_r, ut_r, ct_r, wtag_r, btag_r, w1_r, b1_r, w2_r, b2_r,
              w3_r, b3_r, out_r):
    f32 = jnp.float32
    utf = jnp.maximum(
        jnp.dot(ut_r[...], wtag_r[...], preferred_element_type=f32) + btag_r[...], 0.0)
    ctf = jnp.maximum(
        jnp.dot(ct_r[...], wtag_r[...], preferred_element_type=f32) + btag_r[...], 0.0)
    x = jnp.concatenate([ue_r[...], ce_r[...], utf, ctf], axis=1)
    h = jnp.maximum(jnp.dot(x, w1_r[...], preferred_element_type=f32) + b1_r[...], 0.0)
    h = jnp.maximum(jnp.dot(h, w2_r[...], preferred_element_type=f32) + b2_r[...], 0.0)
    z = jnp.dot(h, w3_r[...], preferred_element_type=f32) + b3_r[...]
    out_r[...] = jax.nn.sigmoid(z)


_mlp = pl.pallas_call(
    _mlp_body,
    grid=(B // BM,),
    in_specs=[
        pl.BlockSpec((BM, D), lambda i: (i, 0)),
        pl.BlockSpec((BM, D), lambda i: (i, 0)),
        pl.BlockSpec((BM, D), lambda i: (i, 0)),
        pl.BlockSpec((BM, D), lambda i: (i, 0)),
        pl.BlockSpec((D, H), lambda i: (0, 0)),
        pl.BlockSpec((1, H), lambda i: (0, 0)),
        pl.BlockSpec((2 * D + 2 * H, 2 * H), lambda i: (0, 0)),
        pl.BlockSpec((1, 2 * H), lambda i: (0, 0)),
        pl.BlockSpec((2 * H, H), lambda i: (0, 0)),
        pl.BlockSpec((1, H), lambda i: (0, 0)),
        pl.BlockSpec((H, 1), lambda i: (0, 0)),
        pl.BlockSpec((1, 1), lambda i: (0, 0)),
    ],
    out_specs=pl.BlockSpec((BM, 1), lambda i: (i, 0)),
    out_shape=jax.ShapeDtypeStruct((B, 1), jnp.float32),
)


def kernel(user_idx, community_idx, user_tag_embedding, community_tag_embedding,
           user_table, community_table, W_tag, b_tag, W1, b1, W2, b2, W3, b3):
    uidx = user_idx.astype(jnp.int32)
    cidx = community_idx.astype(jnp.int32)
    utail = user_table[UBLK * 128:].reshape(-1, 2 * D)
    ctail = community_table[CBLK * 128:].reshape(-1, 2 * D)
    utab2, ctab2 = _build_sc_convert()(user_table.T, community_table.T,
                                       utail, ctail)
    ue, ce = _build_sc_gather()(uidx, cidx, utab2, ctab2)
    return _mlp(ue, ce, user_tag_embedding, community_tag_embedding,
                W_tag, b_tag.reshape(1, H), W1, b1.reshape(1, 2 * H),
                W2, b2.reshape(1, H), W3, b3.reshape(1, 1))


# speedup vs baseline: 4.7175x; 4.7175x over previous
import functools

import jax
import jax.numpy as jnp
from jax import lax
from jax.experimental import pallas as pl
from jax.experimental.pallas import tpu as pltpu
from jax.experimental.pallas import tpu_sc as plsc

B = 16384
D = 64
H = 128
NU = 1000000
NCOMM = 100000

NC = 2
NS = 16
NW = NC * NS
BPW = B // NW
QTR = BPW // 4
HALF = BPW // 2


@functools.lru_cache(maxsize=None)
def _build_sc_gather():
    mesh = plsc.VectorSubcoreMesh(core_axis_name="c", subcore_axis_name="s")

    @functools.partial(
        pl.kernel,
        mesh=mesh,
        out_type=(
            jax.ShapeDtypeStruct((B, D), jnp.float32),
            jax.ShapeDtypeStruct((B, D), jnp.float32),
        ),
        scratch_types=[
            pltpu.VMEM((BPW,), jnp.int32),
            pltpu.VMEM((HALF, D), jnp.float32),
            pltpu.SemaphoreType.DMA,
        ],
    )
    def sc_gather(uidx_hbm, cidx_hbm, utab_hbm, ctab_hbm, uout_hbm, cout_hbm,
                  idx_v, rows_v, sem):
        wid = lax.axis_index("s") * NC + lax.axis_index("c")
        base = wid * BPW

        for idx_hbm, tab_hbm, out_hbm in ((uidx_hbm, utab_hbm, uout_hbm),
                                          (cidx_hbm, ctab_hbm, cout_hbm)):
            pltpu.sync_copy(idx_hbm.at[pl.ds(base, BPW)], idx_v)
            for h in range(2):

                def fire(g, _, tab_hbm=tab_hbm, h=h):
                    vec = idx_v[pl.ds(h * HALF + g * 16, 16)]
                    tvec = lax.shift_right_logical(vec, 3)
                    svec = lax.bitwise_and(vec, 7)
                    for k in range(16):
                        pltpu.async_copy(
                            tab_hbm.at[tvec[k], svec[k]],
                            rows_v.at[g * 16 + k], sem)
                    return _

                lax.fori_loop(0, HALF // 16, fire, None)

                def drain(r, _, tab_hbm=tab_hbm):
                    pltpu.make_async_copy(
                        tab_hbm.at[0, 0], rows_v.at[r], sem).wait()
                    return _

                lax.fori_loop(0, HALF, drain, None)
                pltpu.sync_copy(rows_v, out_hbm.at[pl.ds(base + h * HALF, HALF)])

    return sc_gather


BM = 2048


def _mlp_body(ue_r, ce_r, ut_r, ct_r, wtag_r, btag_r, w1_r, b1_r, w2_r, b2_r,
              w3_r, b3_r, out_r):
    f32 = jnp.float32
    utf = jnp.maximum(
        jnp.dot(ut_r[...], wtag_r[...], preferred_element_type=f32) + btag_r[...], 0.0)
    ctf = jnp.maximum(
        jnp.dot(ct_r[...], wtag_r[...], preferred_element_type=f32) + btag_r[...], 0.0)
    x = jnp.concatenate([ue_r[...], ce_r[...], utf, ctf], axis=1)
    h = jnp.maximum(jnp.dot(x, w1_r[...], preferred_element_type=f32) + b1_r[...], 0.0)
    h = jnp.maximum(jnp.dot(h, w2_r[...], preferred_element_type=f32) + b2_r[...], 0.0)
    z = jnp.dot(h, w3_r[...], preferred_element_type=f32) + b3_r[...]
    out_r[...] = jax.nn.sigmoid(z)


_mlp = pl.pallas_call(
    _mlp_body,
    grid=(B // BM,),
    in_specs=[
        pl.BlockSpec((BM, D), lambda i: (i, 0)),
        pl.BlockSpec((BM, D), lambda i: (i, 0)),
        pl.BlockSpec((BM, D), lambda i: (i, 0)),
        pl.BlockSpec((BM, D), lambda i: (i, 0)),
        pl.BlockSpec((D, H), lambda i: (0, 0)),
        pl.BlockSpec((1, H), lambda i: (0, 0)),
        pl.BlockSpec((2 * D + 2 * H, 2 * H), lambda i: (0, 0)),
        pl.BlockSpec((1, 2 * H), lambda i: (0, 0)),
        pl.BlockSpec((2 * H, H), lambda i: (0, 0)),
        pl.BlockSpec((1, H), lambda i: (0, 0)),
        pl.BlockSpec((H, 1), lambda i: (0, 0)),
        pl.BlockSpec((1, 1), lambda i: (0, 0)),
    ],
    out_specs=pl.BlockSpec((BM, 1), lambda i: (i, 0)),
    out_shape=jax.ShapeDtypeStruct((B, 1), jnp.float32),
)


def kernel(user_idx, community_idx, user_tag_embedding, community_tag_embedding,
           user_table, community_table, W_tag, b_tag, W1, b1, W2, b2, W3, b3):
    uidx = user_idx.astype(jnp.int32)
    cidx = community_idx.astype(jnp.int32)
    utab3 = user_table.reshape(NU // 8, 8, D)
    ctab3 = community_table.reshape(NCOMM // 8, 8, D)
    ue, ce = _build_sc_gather()(uidx, cidx, utab3, ctab3)
    return _mlp(ue, ce, user_tag_embedding, community_tag_embedding,
                W_tag, b_tag.reshape(1, H), W1, b1.reshape(1, 2 * H),
                W2, b2.reshape(1, H), W3, b3.reshape(1, 1))
